# Initial kernel scaffold; baseline (speedup 1.0000x reference)
import jax
import jax.numpy as jnp
from jax.experimental import pallas as pl


def kernel(x):
    raise NotImplementedError("write your pallas kernel here")


# speedup vs baseline: 61.3710x; 61.3710x over previous
import jax
import jax.numpy as jnp
from jax.experimental import pallas as pl


_BJ = 8


def _swap_body(x_ref, o_ref):
    s0 = x_ref[0]
    s1 = x_ref[1]
    col = jax.lax.broadcasted_iota(jnp.int32, s0.shape, dimension=2)
    parity = jax.lax.broadcasted_iota(jnp.int32, s0.shape, dimension=1)
    mask = jnp.logical_and(parity == 1, col % 2 == 1)
    o_ref[0] = jnp.where(mask, s1, s0)
    o_ref[1] = jnp.where(mask, s0, s1)


def kernel(x):
    x4 = x.reshape(2, 2048, 2, 4096)
    grid = (2048 // _BJ,)
    spec = pl.BlockSpec((2, _BJ, 2, 4096), lambda j: (0, j, 0, 0))
    out = pl.pallas_call(
        _swap_body,
        grid=grid,
        in_specs=[spec],
        out_specs=spec,
        out_shape=jax.ShapeDtypeStruct((2, 2048, 2, 4096), jnp.float32),
    )(x4)
    return out.reshape(2, 4096, 4096)


# speedup vs baseline: 63.2754x; 1.0310x over previous
import functools

import jax
import jax.numpy as jnp
from jax import lax
from jax.experimental import pallas as pl
from jax.experimental.pallas import tpu as pltpu
from jax.experimental.pallas import tpu_sc as plsc

_NC = 2
_NS = 16
_NW = _NC * _NS
_J = 2048
_JW = _J // _NW
_B = 2
_NBLK = _JW // _B
_COLS = 4096
_L = 16


def _sc_body(x_hbm, o_hbm, buf):
    wid = lax.axis_index("s") * _NC + lax.axis_index("c")
    base = wid * _JW
    lane = lax.broadcasted_iota(jnp.int32, (_L,), 0)
    odd = (lane % 2) == 1

    def block_body(i, _):
        j0 = base + i * _B
        pltpu.sync_copy(x_hbm.at[0, pl.ds(j0, _B)], buf.at[0])
        pltpu.sync_copy(x_hbm.at[1, pl.ds(j0, _B)], buf.at[1])

        def col_body(c, _):
            c16 = c * _L
            v0 = buf[0, 0, 1, pl.ds(c16, _L)]
            v1 = buf[1, 0, 1, pl.ds(c16, _L)]
            buf[0, 0, 1, pl.ds(c16, _L)] = jnp.where(odd, v1, v0)
            buf[1, 0, 1, pl.ds(c16, _L)] = jnp.where(odd, v0, v1)
            w0 = buf[0, 1, 1, pl.ds(c16, _L)]
            w1 = buf[1, 1, 1, pl.ds(c16, _L)]
            buf[0, 1, 1, pl.ds(c16, _L)] = jnp.where(odd, w1, w0)
            buf[1, 1, 1, pl.ds(c16, _L)] = jnp.where(odd, w0, w1)
            return 0

        lax.fori_loop(0, _COLS // _L, col_body, 0)
        pltpu.sync_copy(buf.at[0], o_hbm.at[0, pl.ds(j0, _B)])
        pltpu.sync_copy(buf.at[1], o_hbm.at[1, pl.ds(j0, _B)])
        return 0

    lax.fori_loop(0, _NBLK, block_body, 0)


_sc_kernel = functools.partial(
    pl.kernel,
    mesh=plsc.VectorSubcoreMesh(core_axis_name="c", subcore_axis_name="s"),
    out_type=jax.ShapeDtypeStruct((2, _J, 2, _COLS), jnp.float32),
    scratch_types=[pltpu.VMEM((2, _B, 2, _COLS), jnp.float32)],
)(_sc_body)


def kernel(x):
    x4 = x.reshape(2, _J, 2, _COLS)
    return _sc_kernel(x4).reshape(2, 4096, 4096)


# speedup vs baseline: 71.3674x; 1.1279x over previous
import functools

import jax
import jax.numpy as jnp
from jax import lax
from jax.experimental import pallas as pl
from jax.experimental.pallas import tpu as pltpu
from jax.experimental.pallas import tpu_sc as plsc

_NC = 2
_NS = 16
_NW = _NC * _NS
_J = 2048
_JW = _J // _NW
_B = 4
_NBLK = _JW // _B
_COLS = 4096
_L = 16


def _sc_body(x_hbm, o_hbm, buf, sem):
    wid = lax.axis_index("s") * _NC + lax.axis_index("c")
    base = wid * _JW
    lane = lax.broadcasted_iota(jnp.int32, (_L,), 0)
    odd = (lane % 2) == 1

    def block_body(i, _):
        j0 = base + i * _B
        pltpu.async_copy(x_hbm.at[0, pl.ds(j0, _B)], buf.at[0], sem)
        pltpu.async_copy(x_hbm.at[1, pl.ds(j0, _B)], buf.at[1], sem)
        pltpu.make_async_copy(x_hbm.at[0, pl.ds(j0, _B)], buf.at[0], sem).wait()
        pltpu.make_async_copy(x_hbm.at[1, pl.ds(j0, _B)], buf.at[1], sem).wait()

        def col_body(c, _):
            c16 = c * _L
            for r in range(_B):
                v0 = buf[0, r, 1, pl.ds(c16, _L)]
                v1 = buf[1, r, 1, pl.ds(c16, _L)]
                buf[0, r, 1, pl.ds(c16, _L)] = jnp.where(odd, v1, v0)
                buf[1, r, 1, pl.ds(c16, _L)] = jnp.where(odd, v0, v1)
            return 0

        lax.fori_loop(0, _COLS // _L, col_body, 0)
        pltpu.async_copy(buf.at[0], o_hbm.at[0, pl.ds(j0, _B)], sem)
        pltpu.async_copy(buf.at[1], o_hbm.at[1, pl.ds(j0, _B)], sem)
        pltpu.make_async_copy(buf.at[0], o_hbm.at[0, pl.ds(j0, _B)], sem).wait()
        pltpu.make_async_copy(buf.at[1], o_hbm.at[1, pl.ds(j0, _B)], sem).wait()
        return 0

    lax.fori_loop(0, _NBLK, block_body, 0)


_sc_kernel = functools.partial(
    pl.kernel,
    mesh=plsc.VectorSubcoreMesh(core_axis_name="c", subcore_axis_name="s"),
    out_type=jax.ShapeDtypeStruct((2, _J, 2, _COLS), jnp.float32),
    scratch_types=[
        pltpu.VMEM((2, _B, 2, _COLS), jnp.float32),
        pltpu.SemaphoreType.DMA,
    ],
)(_sc_body)


def kernel(x):
    x4 = x.reshape(2, _J, 2, _COLS)
    return _sc_kernel(x4).reshape(2, 4096, 4096)


# speedup vs baseline: 194.1857x; 2.7209x over previous
import functools

import jax
import jax.numpy as jnp
from jax import lax
from jax.experimental import pallas as pl
from jax.experimental.pallas import tpu as pltpu
from jax.experimental.pallas import tpu_sc as plsc

_NC = 2
_NS = 16
_NW = _NC * _NS
_ROWS = 4096
_RW = _ROWS // _NW
_R = 8
_NBLK = _RW // _R
_COLS = 4096
_L = 16


def _sc_body(x_hbm, o_hbm, buf, sem):
    wid = lax.axis_index("s") * _NC + lax.axis_index("c")
    base = wid * _RW
    lane = lax.broadcasted_iota(jnp.int32, (_L,), 0)
    odd = (lane % 2) == 1

    def block_body(i, _):
        r0 = base + i * _R
        pltpu.async_copy(x_hbm.at[0, pl.ds(r0, _R)], buf.at[0], sem)
        pltpu.async_copy(x_hbm.at[1, pl.ds(r0, _R)], buf.at[1], sem)
        pltpu.make_async_copy(x_hbm.at[0, pl.ds(r0, _R)], buf.at[0], sem).wait()
        pltpu.make_async_copy(x_hbm.at[1, pl.ds(r0, _R)], buf.at[1], sem).wait()

        def col_body(c, _):
            c16 = c * _L
            for r in range(1, _R, 2):
                v0 = buf[0, r, pl.ds(c16, _L)]
                v1 = buf[1, r, pl.ds(c16, _L)]
                buf[0, r, pl.ds(c16, _L)] = jnp.where(odd, v1, v0)
                buf[1, r, pl.ds(c16, _L)] = jnp.where(odd, v0, v1)
            return 0

        lax.fori_loop(0, _COLS // _L, col_body, 0)
        pltpu.async_copy(buf.at[0], o_hbm.at[0, pl.ds(r0, _R)], sem)
        pltpu.async_copy(buf.at[1], o_hbm.at[1, pl.ds(r0, _R)], sem)
        pltpu.make_async_copy(buf.at[0], o_hbm.at[0, pl.ds(r0, _R)], sem).wait()
        pltpu.make_async_copy(buf.at[1], o_hbm.at[1, pl.ds(r0, _R)], sem).wait()
        return 0

    lax.fori_loop(0, _NBLK, block_body, 0)


kernel = functools.partial(
    pl.kernel,
    mesh=plsc.VectorSubcoreMesh(core_axis_name="c", subcore_axis_name="s"),
    out_type=jax.ShapeDtypeStruct((2, _ROWS, _COLS), jnp.float32),
    scratch_types=[
        pltpu.VMEM((2, _R, _COLS), jnp.float32),
        pltpu.SemaphoreType.DMA,
    ],
)(_sc_body)


# speedup vs baseline: 256.1841x; 1.3193x over previous
import functools

import jax
import jax.numpy as jnp
from jax import lax
from jax.experimental import pallas as pl
from jax.experimental.pallas import tpu as pltpu
from jax.experimental.pallas import tpu_sc as plsc

_NC = 2
_NS = 16
_NW = _NC * _NS
_ROWS = 4096
_RW = _ROWS // _NW
_R = 2
_NBLK = _RW // _R
_NSLOT = 4
_COLS = 4096
_L = 16


def _sc_body(x_hbm, o_hbm, buf, *sems):
    sin, sout = sems[:_NSLOT], sems[_NSLOT:]
    wid = lax.axis_index("s") * _NC + lax.axis_index("c")
    base = wid * _RW
    lane = lax.broadcasted_iota(jnp.int32, (_L,), 0)
    odd = (lane % 2) == 1

    def start_in(slot, i):
        r0 = base + i * _R
        pltpu.async_copy(x_hbm.at[0, pl.ds(r0, _R)], buf.at[slot, 0], sin[slot])
        pltpu.async_copy(x_hbm.at[1, pl.ds(r0, _R)], buf.at[slot, 1], sin[slot])

    def wait_in(slot, i):
        r0 = base + i * _R
        pltpu.make_async_copy(
            x_hbm.at[0, pl.ds(r0, _R)], buf.at[slot, 0], sin[slot]).wait()
        pltpu.make_async_copy(
            x_hbm.at[1, pl.ds(r0, _R)], buf.at[slot, 1], sin[slot]).wait()

    def start_out(slot, i):
        r0 = base + i * _R
        pltpu.async_copy(buf.at[slot, 0], o_hbm.at[0, pl.ds(r0, _R)], sout[slot])
        pltpu.async_copy(buf.at[slot, 1], o_hbm.at[1, pl.ds(r0, _R)], sout[slot])

    def wait_out(slot, i):
        r0 = base + i * _R
        pltpu.make_async_copy(
            buf.at[slot, 0], o_hbm.at[0, pl.ds(r0, _R)], sout[slot]).wait()
        pltpu.make_async_copy(
            buf.at[slot, 1], o_hbm.at[1, pl.ds(r0, _R)], sout[slot]).wait()

    def compute(slot):
        def col_body(c, _):
            c16 = c * _L
            for r in range(1, _R, 2):
                v0 = buf[slot, 0, r, pl.ds(c16, _L)]
                v1 = buf[slot, 1, r, pl.ds(c16, _L)]
                buf[slot, 0, r, pl.ds(c16, _L)] = jnp.where(odd, v1, v0)
                buf[slot, 1, r, pl.ds(c16, _L)] = jnp.where(odd, v0, v1)
            return 0

        lax.fori_loop(0, _COLS // _L, col_body, 0)

    start_in(0, 0)
    start_in(1, 1)

    def quad_body(k, _):
        for u in range(_NSLOT):
            i = k * _NSLOT + u
            pslot = (u + 2) % _NSLOT

            @pl.when(i >= 2)
            def _():
                wait_out(pslot, i - 2)

            @pl.when(i + 2 < _NBLK)
            def _():
                start_in(pslot, i + 2)

            wait_in(u, i)
            compute(u)
            start_out(u, i)
        return 0

    lax.fori_loop(0, _NBLK // _NSLOT, quad_body, 0)
    wait_out((_NBLK - 2) % _NSLOT, _NBLK - 2)
    wait_out((_NBLK - 1) % _NSLOT, _NBLK - 1)


kernel = functools.partial(
    pl.kernel,
    mesh=plsc.VectorSubcoreMesh(core_axis_name="c", subcore_axis_name="s"),
    out_type=jax.ShapeDtypeStruct((2, _ROWS, _COLS), jnp.float32),
    scratch_types=[pltpu.VMEM((_NSLOT, 2, _R, _COLS), jnp.float32)]
    + [pltpu.SemaphoreType.DMA] * (2 * _NSLOT),
)(_sc_body)


# speedup vs baseline: 289.2143x; 1.1289x over previous
import functools

import jax
import jax.numpy as jnp
from jax import lax
from jax.experimental import pallas as pl
from jax.experimental.pallas import tpu as pltpu
from jax.experimental.pallas import tpu_sc as plsc

_NC = 2
_NS = 16
_NW = _NC * _NS
_ROWS = 4096
_RW = _ROWS // _NW
_R = 2
_NBLK = _RW // _R
_NSLOT = 4
_COLS = 4096
_L = 16


def _sc_body(x_hbm, o_hbm, buf, *sems):
    sin, sout = sems[:_NSLOT], sems[_NSLOT:]
    wid = lax.axis_index("s") * _NC + lax.axis_index("c")
    base = wid * _RW
    lane = lax.broadcasted_iota(jnp.int32, (_L,), 0)
    odd = (lane % 2) == 1

    def start_in(slot, i):
        r0 = base + i * _R
        pltpu.async_copy(x_hbm.at[0, pl.ds(r0, _R)], buf.at[slot, 0], sin[slot])
        pltpu.async_copy(x_hbm.at[1, pl.ds(r0, _R)], buf.at[slot, 1], sin[slot])

    def wait_in(slot, i):
        r0 = base + i * _R
        pltpu.make_async_copy(
            x_hbm.at[0, pl.ds(r0, _R)], buf.at[slot, 0], sin[slot]).wait()
        pltpu.make_async_copy(
            x_hbm.at[1, pl.ds(r0, _R)], buf.at[slot, 1], sin[slot]).wait()

    def start_out(slot, i):
        r0 = base + i * _R
        pltpu.async_copy(buf.at[slot, 0], o_hbm.at[0, pl.ds(r0, _R)], sout[slot])
        pltpu.async_copy(buf.at[slot, 1], o_hbm.at[1, pl.ds(r0, _R)], sout[slot])

    def wait_out(slot, i):
        r0 = base + i * _R
        pltpu.make_async_copy(
            buf.at[slot, 0], o_hbm.at[0, pl.ds(r0, _R)], sout[slot]).wait()
        pltpu.make_async_copy(
            buf.at[slot, 1], o_hbm.at[1, pl.ds(r0, _R)], sout[slot]).wait()

    def compute(slot):
        unroll = 8

        def col_body(c, _):
            for v in range(unroll):
                c16 = (c * unroll + v) * _L
                for r in range(1, _R, 2):
                    v0 = buf[slot, 0, r, pl.ds(c16, _L)]
                    v1 = buf[slot, 1, r, pl.ds(c16, _L)]
                    buf[slot, 0, r, pl.ds(c16, _L)] = jnp.where(odd, v1, v0)
                    buf[slot, 1, r, pl.ds(c16, _L)] = jnp.where(odd, v0, v1)
            return 0

        lax.fori_loop(0, _COLS // _L // unroll, col_body, 0)

    start_in(0, 0)
    start_in(1, 1)

    def quad_body(k, _):
        for u in range(_NSLOT):
            i = k * _NSLOT + u
            pslot = (u + 2) % _NSLOT

            @pl.when(i >= 2)
            def _():
                wait_out(pslot, i - 2)

            @pl.when(i + 2 < _NBLK)
            def _():
                start_in(pslot, i + 2)

            wait_in(u, i)
            compute(u)
            start_out(u, i)
        return 0

    lax.fori_loop(0, _NBLK // _NSLOT, quad_body, 0)
    wait_out((_NBLK - 2) % _NSLOT, _NBLK - 2)
    wait_out((_NBLK - 1) % _NSLOT, _NBLK - 1)


kernel = functools.partial(
    pl.kernel,
    mesh=plsc.VectorSubcoreMesh(core_axis_name="c", subcore_axis_name="s"),
    out_type=jax.ShapeDtypeStruct((2, _ROWS, _COLS), jnp.float32),
    scratch_types=[pltpu.VMEM((_NSLOT, 2, _R, _COLS), jnp.float32)]
    + [pltpu.SemaphoreType.DMA] * (2 * _NSLOT),
)(_sc_body)


# speedup vs baseline: 289.3676x; 1.0005x over previous
import functools

import jax
import jax.numpy as jnp
from jax import lax
from jax.experimental import pallas as pl
from jax.experimental.pallas import tpu as pltpu
from jax.experimental.pallas import tpu_sc as plsc

_NC = 2
_NS = 16
_NW = _NC * _NS
_ROWS = 4096
_RW = _ROWS // _NW
_R = 2
_NBLK = _RW // _R
_NSLOT = 4
_COLS = 4096
_L = 16


def _sc_body(x_hbm, o_hbm, buf, *sems):
    sin, sout = sems[:_NSLOT], sems[_NSLOT:]
    wid = lax.axis_index("s") * _NC + lax.axis_index("c")
    base = wid * _RW
    lane = lax.broadcasted_iota(jnp.int32, (_L,), 0)
    odd = (lane % 2) == 1

    def start_in(slot, i):
        r0 = base + i * _R
        pltpu.async_copy(x_hbm.at[0, pl.ds(r0, _R)], buf.at[slot, 0], sin[slot])
        pltpu.async_copy(x_hbm.at[1, pl.ds(r0, _R)], buf.at[slot, 1], sin[slot])

    def wait_in(slot, i):
        r0 = base + i * _R
        pltpu.make_async_copy(
            x_hbm.at[0, pl.ds(r0, _R)], buf.at[slot, 0], sin[slot]).wait()
        pltpu.make_async_copy(
            x_hbm.at[1, pl.ds(r0, _R)], buf.at[slot, 1], sin[slot]).wait()

    def start_out(slot, i):
        r0 = base + i * _R
        pltpu.async_copy(buf.at[slot, 0], o_hbm.at[0, pl.ds(r0, _R)], sout[slot])
        pltpu.async_copy(buf.at[slot, 1], o_hbm.at[1, pl.ds(r0, _R)], sout[slot])

    def wait_out(slot, i):
        r0 = base + i * _R
        pltpu.make_async_copy(
            buf.at[slot, 0], o_hbm.at[0, pl.ds(r0, _R)], sout[slot]).wait()
        pltpu.make_async_copy(
            buf.at[slot, 1], o_hbm.at[1, pl.ds(r0, _R)], sout[slot]).wait()

    def compute(slot):
        @plsc.parallel_loop(0, _COLS // _L, 1, unroll=8)
        def col_body(c):
            c16 = c * _L
            for r in range(1, _R, 2):
                v0 = buf[slot, 0, r, pl.ds(c16, _L)]
                v1 = buf[slot, 1, r, pl.ds(c16, _L)]
                buf[slot, 0, r, pl.ds(c16, _L)] = jnp.where(odd, v1, v0)
                buf[slot, 1, r, pl.ds(c16, _L)] = jnp.where(odd, v0, v1)

    start_in(0, 0)
    start_in(1, 1)

    def quad_body(k, _):
        for u in range(_NSLOT):
            i = k * _NSLOT + u
            pslot = (u + 2) % _NSLOT

            @pl.when(i >= 2)
            def _():
                wait_out(pslot, i - 2)

            @pl.when(i + 2 < _NBLK)
            def _():
                start_in(pslot, i + 2)

            wait_in(u, i)
            compute(u)
            start_out(u, i)
        return 0

    lax.fori_loop(0, _NBLK // _NSLOT, quad_body, 0)
    wait_out((_NBLK - 2) % _NSLOT, _NBLK - 2)
    wait_out((_NBLK - 1) % _NSLOT, _NBLK - 1)


kernel = functools.partial(
    pl.kernel,
    mesh=plsc.VectorSubcoreMesh(core_axis_name="c", subcore_axis_name="s"),
    out_type=jax.ShapeDtypeStruct((2, _ROWS, _COLS), jnp.float32),
    scratch_types=[pltpu.VMEM((_NSLOT, 2, _R, _COLS), jnp.float32)]
    + [pltpu.SemaphoreType.DMA] * (2 * _NSLOT),
)(_sc_body)
